# Initial kernel scaffold; baseline (speedup 1.0000x reference)
import jax
import jax.numpy as jnp
from jax.experimental import pallas as pl


def kernel(x, edge_index, batch, params):
    raise NotImplementedError("write your pallas kernel here")


# speedup vs baseline: 4.5843x; 4.5843x over previous
import functools

import jax
import jax.numpy as jnp
import numpy as np
from jax import lax
from jax.experimental import pallas as pl
from jax.experimental.pallas import tpu as pltpu
from jax.experimental.pallas import tpu_sc as plsc

N = 10000
E = 320000
D = 128
H = 128
OUT = 256
L = 5
B = 64
BN_EPS = 1e-5

NC = 2
NS = 16
NW = NC * NS
CHUNK = 128
EPW = E // NW
EPW_CH = (EPW + CHUNK - 1) // CHUNK
EPW_PAD = EPW_CH * CHUNK - EPW
NPAD = 10240
RPS = NPAD // NS

R = 1000
G = N // R


def _sc_scatter_add(h, src3, dst3, zeros_blk):
    mesh = plsc.VectorSubcoreMesh(core_axis_name="c", subcore_axis_name="s")

    @functools.partial(
        pl.kernel,
        mesh=mesh,
        out_type=jax.ShapeDtypeStruct((NC, NPAD, D), jnp.float32),
        scratch_types=[
            pltpu.VMEM((EPW_CH, CHUNK), jnp.int32),
            pltpu.VMEM((EPW_CH, CHUNK), jnp.int32),
            pltpu.VMEM((CHUNK, D), jnp.float32),
            pltpu.VMEM_SHARED((NPAD, D), jnp.float32),
            pltpu.SemaphoreType.DMA,
        ],
    )
    def k(
        h_hbm, src_hbm, dst_hbm, z_hbm, out_hbm,
        src_v, dst_v, rows_v, agg_sh, sem,
    ):
        c = lax.axis_index("c")
        s = lax.axis_index("s")
        wid = s * NC + c

        pltpu.sync_copy(z_hbm, rows_v)

        def zc(j, carry):
            pltpu.sync_copy(rows_v, agg_sh.at[pl.ds(s * RPS + j * CHUNK, CHUNK)])
            return carry

        lax.fori_loop(0, RPS // CHUNK, zc, 0)
        plsc.subcore_barrier()

        pltpu.sync_copy(src_hbm.at[wid], src_v)
        pltpu.sync_copy(dst_hbm.at[wid], dst_v)

        def edge_chunk(j, carry):
            pltpu.async_copy(h_hbm.at[src_v.at[j]], rows_v, sem).wait()
            pltpu.sync_copy(rows_v, agg_sh.at[dst_v.at[j]], add=True)
            return carry

        lax.fori_loop(0, EPW_CH, edge_chunk, 0)
        plsc.subcore_barrier()

        pltpu.sync_copy(
            agg_sh.at[pl.ds(s * RPS, RPS)], out_hbm.at[c, pl.ds(s * RPS, RPS)]
        )

    return k(h, src3, dst3, zeros_blk)


def _mlp_mid(h, aggs, W1, b1, W2, b2):
    def body(h_ref, a_ref, w1_ref, b1_ref, w2_ref, b2_ref, out_ref):
        z = h_ref[...] + a_ref[0] + a_ref[1]
        z = jnp.dot(z, w1_ref[...], preferred_element_type=jnp.float32) + b1_ref[...]
        z = jnp.maximum(z, 0.0)
        z = jnp.dot(z, w2_ref[...], preferred_element_type=jnp.float32) + b2_ref[...]
        z = jnp.maximum(z, 0.0)
        out_ref[...] = h_ref[...] + z

    return pl.pallas_call(
        body,
        grid=(G,),
        in_specs=[
            pl.BlockSpec((R, D), lambda i: (i, 0)),
            pl.BlockSpec((NC, R, D), lambda i: (0, i, 0)),
            pl.BlockSpec((D, H), lambda i: (0, 0)),
            pl.BlockSpec((1, H), lambda i: (0, 0)),
            pl.BlockSpec((H, H), lambda i: (0, 0)),
            pl.BlockSpec((1, H), lambda i: (0, 0)),
        ],
        out_specs=pl.BlockSpec((R, D), lambda i: (i, 0)),
        out_shape=jax.ShapeDtypeStruct((N, D), jnp.float32),
    )(h, aggs, W1, b1, W2, b2)


def _mlp_last(h, aggs, W1, b1, W2, b2, batch2, Wout, bout):
    def body(
        h_ref, a_ref, w1_ref, b1_ref, w2_ref, b2_ref, bt_ref, wo_ref, bo_ref,
        out_ref, g_ref, sums_ref, cnts_ref,
    ):
        i = pl.program_id(0)
        z = h_ref[...] + a_ref[0] + a_ref[1]
        z = jnp.dot(z, w1_ref[...], preferred_element_type=jnp.float32) + b1_ref[...]
        z = jnp.maximum(z, 0.0)
        z = jnp.dot(z, w2_ref[...], preferred_element_type=jnp.float32) + b2_ref[...]
        z = jnp.maximum(z, 0.0)
        hnew = h_ref[...] + z
        out_ref[...] = hnew

        onehot = (
            bt_ref[...] == lax.broadcasted_iota(jnp.int32, (R, B), 1)
        ).astype(jnp.float32)
        part = lax.dot_general(
            onehot, hnew, (((0,), (0,)), ((), ())),
            preferred_element_type=jnp.float32,
        )
        cnt = lax.dot_general(
            onehot, jnp.ones((R, 1), jnp.float32), (((0,), (0,)), ((), ())),
            preferred_element_type=jnp.float32,
        )

        @pl.when(i == 0)
        def _():
            sums_ref[...] = part
            cnts_ref[...] = cnt

        @pl.when(i > 0)
        def _():
            sums_ref[...] += part
            cnts_ref[...] += cnt

        @pl.when(i == G - 1)
        def _():
            mean = sums_ref[...] / jnp.maximum(cnts_ref[...], 1.0)
            g_ref[...] = (
                jnp.dot(mean, wo_ref[...], preferred_element_type=jnp.float32)
                + bo_ref[...]
            )

    return pl.pallas_call(
        body,
        grid=(G,),
        in_specs=[
            pl.BlockSpec((R, D), lambda i: (i, 0)),
            pl.BlockSpec((NC, R, D), lambda i: (0, i, 0)),
            pl.BlockSpec((D, H), lambda i: (0, 0)),
            pl.BlockSpec((1, H), lambda i: (0, 0)),
            pl.BlockSpec((H, H), lambda i: (0, 0)),
            pl.BlockSpec((1, H), lambda i: (0, 0)),
            pl.BlockSpec((R, 1), lambda i: (i, 0)),
            pl.BlockSpec((H, OUT), lambda i: (0, 0)),
            pl.BlockSpec((1, OUT), lambda i: (0, 0)),
        ],
        out_specs=[
            pl.BlockSpec((R, D), lambda i: (i, 0)),
            pl.BlockSpec((B, OUT), lambda i: (0, 0)),
        ],
        out_shape=[
            jax.ShapeDtypeStruct((N, D), jnp.float32),
            jax.ShapeDtypeStruct((B, OUT), jnp.float32),
        ],
        scratch_shapes=[
            pltpu.VMEM((B, H), jnp.float32),
            pltpu.VMEM((B, 1), jnp.float32),
        ],
    )(h, aggs, W1, b1, W2, b2, batch2, Wout, bout)


def kernel(x, edge_index, batch, params):
    inv = np.float32(1.0 / np.sqrt(1.0 + BN_EPS))

    src = edge_index[0].reshape(NW, EPW)
    dst = edge_index[1].reshape(NW, EPW)
    src3 = jnp.concatenate(
        [src, jnp.zeros((NW, EPW_PAD), jnp.int32)], axis=1
    ).reshape(NW, EPW_CH, CHUNK)
    dst3 = jnp.concatenate(
        [dst, jnp.full((NW, EPW_PAD), N, jnp.int32)], axis=1
    ).reshape(NW, EPW_CH, CHUNK)
    zeros_blk = jnp.zeros((CHUNK, D), jnp.float32)
    batch2 = batch.reshape(N, 1)

    Ws1, bs1, Ws2, bs2 = [], [], [], []
    for i in range(L):
        g1 = params["l%d_g1" % i] * inv
        Ws1.append(params["l%d_W1" % i] * g1[None, :])
        bs1.append((params["l%d_b1" % i] * g1 + params["l%d_be1" % i]).reshape(1, H))
        g2 = params["l%d_bng" % i] * inv
        Ws2.append(params["l%d_W2" % i] * g2[None, :])
        bs2.append((params["l%d_b2" % i] * g2 + params["l%d_bnb" % i]).reshape(1, H))

    h = x
    for i in range(L):
        aggs = _sc_scatter_add(h, src3, dst3, zeros_blk)
        if i < L - 1:
            h = _mlp_mid(h, aggs, Ws1[i], bs1[i], Ws2[i], bs2[i])
        else:
            h, graph = _mlp_last(
                h, aggs, Ws1[i], bs1[i], Ws2[i], bs2[i], batch2,
                params["Wout"], params["bout"].reshape(1, OUT),
            )
    return (graph, h)
